# baseline (speedup 1.0000x reference)
import functools

import jax
import jax.numpy as jnp
from jax import lax
from jax.experimental import pallas as pl
from jax.experimental.pallas import tpu as pltpu
from jax.experimental.pallas import tpu_sc as plsc

_B = 4
_N = 2048
_NQ = _B * _N
_L = 16


def _ladder(m1, m2, m3, d):
    t = jnp.maximum(m1, d)
    m1 = jnp.minimum(m1, d)
    t2 = jnp.maximum(m2, t)
    m2 = jnp.minimum(m2, t)
    m3 = jnp.minimum(m3, t2)
    return m1, m2, m3


def _sc_knn(p1t, p2t, c1t, out, kx, ky, kz, cr, cg, cb, qx, qy, qz,
            dbuf, ovr, ovg, ovb):
    nw = 32
    qpw = _NQ // nw
    wid = lax.axis_index("c") * 16 + lax.axis_index("s")
    b = wid // (_N // qpw)
    qoff = (wid % (_N // qpw)) * qpw

    pltpu.sync_copy(p1t.at[pl.ds((b * 3 + 0) * _N, _N)], kx)
    pltpu.sync_copy(p1t.at[pl.ds((b * 3 + 1) * _N, _N)], ky)
    pltpu.sync_copy(p1t.at[pl.ds((b * 3 + 2) * _N, _N)], kz)
    pltpu.sync_copy(c1t.at[pl.ds((b * 3 + 0) * _N, _N)], cr)
    pltpu.sync_copy(c1t.at[pl.ds((b * 3 + 1) * _N, _N)], cg)
    pltpu.sync_copy(c1t.at[pl.ds((b * 3 + 2) * _N, _N)], cb)
    pltpu.sync_copy(p2t.at[pl.ds((b * 3 + 0) * _N + qoff, qpw)], qx)
    pltpu.sync_copy(p2t.at[pl.ds((b * 3 + 1) * _N + qoff, qpw)], qy)
    pltpu.sync_copy(p2t.at[pl.ds((b * 3 + 2) * _N + qoff, qpw)], qz)

    inf16 = jnp.full((_L,), jnp.inf, jnp.float32)
    zero16 = jnp.zeros((_L,), jnp.float32)
    one16 = jnp.ones((_L,), jnp.float32)

    lane = lax.iota(jnp.int32, _L)
    ninf16 = jnp.full((_L,), -jnp.inf, jnp.float32)

    _dn = lax.GatherDimensionNumbers(
        offset_dims=(), collapsed_slice_dims=(0,), start_index_map=(0,))

    def _shuf(v, idx):
        return lax.gather(v, idx[:, None], _dn, (1,),
                          mode=lax.GatherScatterMode.PROMISE_IN_BOUNDS)

    def _bfly(v, op):
        for s in (8, 4, 2, 1):
            v = op(v, _shuf(v, lane ^ s))
        return v

    def query_body(j, _):
        g = j // _L
        sel = lane == (j % _L)
        qxb = _bfly(jnp.where(sel, qx[pl.ds(g * _L, _L)], ninf16), jnp.maximum)
        qyb = _bfly(jnp.where(sel, qy[pl.ds(g * _L, _L)], ninf16), jnp.maximum)
        qzb = _bfly(jnp.where(sel, qz[pl.ds(g * _L, _L)], ninf16), jnp.maximum)

        def p1_body(c, ms):
            m1a, m2a, m3a, m1b, m2b, m3b = ms
            off = c * 32
            dxa = kx[pl.ds(off, _L)] - qxb
            dya = ky[pl.ds(off, _L)] - qyb
            dza = kz[pl.ds(off, _L)] - qzb
            da = (dxa * dxa + dya * dya) + dza * dza
            dbuf[pl.ds(off, _L)] = da
            dxb = kx[pl.ds(off + _L, _L)] - qxb
            dyb = ky[pl.ds(off + _L, _L)] - qyb
            dzb = kz[pl.ds(off + _L, _L)] - qzb
            db = (dxb * dxb + dyb * dyb) + dzb * dzb
            dbuf[pl.ds(off + _L, _L)] = db
            m1a, m2a, m3a = _ladder(m1a, m2a, m3a, da)
            m1b, m2b, m3b = _ladder(m1b, m2b, m3b, db)
            return (m1a, m2a, m3a, m1b, m2b, m3b)

        m1, m2, m3, m1b, m2b, m3b = lax.fori_loop(
            0, _N // 32, p1_body, (inf16,) * 6)
        for v in (m1b, m2b, m3b):
            m1, m2, m3 = _ladder(m1, m2, m3, v)
        r1 = _bfly(m1, jnp.minimum)
        e1 = m1 == r1
        m1 = jnp.where(e1, m2, m1)
        m2 = jnp.where(e1, m3, m2)
        r2 = _bfly(m1, jnp.minimum)
        e2 = m1 == r2
        m1 = jnp.where(e2, m2, m1)
        thr = _bfly(m1, jnp.minimum)

        def p2_body(c, acc):
            ar, ag, ab, cn = acc
            off = c * 32
            da = dbuf[pl.ds(off, _L)]
            sa = da <= thr
            ar = ar + jnp.where(sa, cr[pl.ds(off, _L)], zero16)
            ag = ag + jnp.where(sa, cg[pl.ds(off, _L)], zero16)
            ab = ab + jnp.where(sa, cb[pl.ds(off, _L)], zero16)
            cn = cn + jnp.where(sa, one16, zero16)
            db2 = dbuf[pl.ds(off + _L, _L)]
            sb = db2 <= thr
            ar = ar + jnp.where(sb, cr[pl.ds(off + _L, _L)], zero16)
            ag = ag + jnp.where(sb, cg[pl.ds(off + _L, _L)], zero16)
            ab = ab + jnp.where(sb, cb[pl.ds(off + _L, _L)], zero16)
            cn = cn + jnp.where(sb, one16, zero16)
            return (ar, ag, ab, cn)

        ar, ag, ab, cn = lax.fori_loop(
            0, _N // 32, p2_body, (zero16,) * 4)
        inv = one16 / _bfly(cn, jnp.add)
        ovr[pl.ds(g * _L, _L)] = jnp.where(sel, _bfly(ar, jnp.add) * inv,
                                           ovr[pl.ds(g * _L, _L)])
        ovg[pl.ds(g * _L, _L)] = jnp.where(sel, _bfly(ag, jnp.add) * inv,
                                           ovg[pl.ds(g * _L, _L)])
        ovb[pl.ds(g * _L, _L)] = jnp.where(sel, _bfly(ab, jnp.add) * inv,
                                           ovb[pl.ds(g * _L, _L)])
        return 0

    lax.fori_loop(0, qpw, query_body, 0)

    base = b * _N + qoff
    pltpu.sync_copy(ovr, out.at[pl.ds(0 * _NQ + base, qpw)])
    pltpu.sync_copy(ovg, out.at[pl.ds(1 * _NQ + base, qpw)])
    pltpu.sync_copy(ovb, out.at[pl.ds(2 * _NQ + base, qpw)])


def kernel(points1, points2, colors1):
    f32 = jnp.float32
    p1t = jnp.transpose(points1, (0, 2, 1)).reshape(_B * 3 * _N)
    p2t = jnp.transpose(points2, (0, 2, 1)).reshape(_B * 3 * _N)
    c1t = jnp.transpose(colors1, (0, 2, 1)).reshape(_B * 3 * _N)

    mesh = plsc.VectorSubcoreMesh(core_axis_name="c", subcore_axis_name="s")
    sc = functools.partial(
        pl.kernel,
        mesh=mesh,
        out_type=jax.ShapeDtypeStruct((3 * _NQ,), f32),
        scratch_types=[
            pltpu.VMEM((_N,), f32),
            pltpu.VMEM((_N,), f32),
            pltpu.VMEM((_N,), f32),
            pltpu.VMEM((_N,), f32),
            pltpu.VMEM((_N,), f32),
            pltpu.VMEM((_N,), f32),
            pltpu.VMEM((_NQ // 32,), f32),
            pltpu.VMEM((_NQ // 32,), f32),
            pltpu.VMEM((_NQ // 32,), f32),
            pltpu.VMEM((_N,), f32),
            pltpu.VMEM((_NQ // 32,), f32),
            pltpu.VMEM((_NQ // 32,), f32),
            pltpu.VMEM((_NQ // 32,), f32),
        ],
    )(_sc_knn)
    out_t = sc(p1t, p2t, c1t)
    return jnp.transpose(out_t.reshape(3, _B, _N), (1, 2, 0))
